# baseline (speedup 1.0000x reference)
import jax
import jax.numpy as jnp
from jax import lax
from jax.experimental import pallas as pl
from jax.experimental.pallas import tpu as pltpu
from jax.experimental.pallas import tpu_sc as plsc

L = 200
N = 1024
D = 64
NUM_CORES = 2
NUM_SUBCORES = 16
NW = NUM_CORES * NUM_SUBCORES
NBLK = N // NW
LCH = 4
CH_ROWS = NBLK * LCH
NCH = L // LCH
PAIRS = NCH // 2
LANES = 16
KG = D // LANES


def _sc_body(x_hbm, st_hbm, tok_hbm, seg_hbm, pos_hbm, out_hbm,
             xr, sr, pos_v, segt, dline, idxc, tok0, tok1, ob0, ob1,
             c00, c01, semg0, semg1, semo0, semo1):
    toks = (tok0, tok1)
    obs = (ob0, ob1)
    c0s = (c00, c01)
    semgs = (semg0, semg1)
    semos = (semo0, semo1)

    wid = lax.axis_index("s") * NUM_CORES + lax.axis_index("c")
    n0 = wid * NBLK
    pltpu.sync_copy(x_hbm.at[pl.ds(n0, NBLK)], xr)
    pltpu.sync_copy(st_hbm.at[pl.ds(n0, NBLK)], sr)
    pltpu.sync_copy(pos_hbm.at[pl.ds(0, L)], pos_v)
    pltpu.sync_copy(seg_hbm, segt)
    for k in range(KG):
        ksl = pl.ds(k * LANES, LANES)
        dline[0, ksl] = segt[1, ksl] - segt[0, ksl]

    iota = lax.iota(jnp.int32, LANES)
    nvecs = tuple(iota + (LANES * h) for h in range(NBLK // LANES))

    def issue(g, b):
        l0 = g * LCH
        for s in range(LCH):
            lvec = jnp.full((LANES,), l0 + s, dtype=jnp.int32)
            for h, nvec in enumerate(nvecs):
                v = plsc.load_gather(xr, [nvec, lvec])
                idxc[b, pl.ds(s * NBLK + h * LANES, LANES)] = v
        pltpu.async_copy(tok_hbm.at[idxc.at[b]], toks[b], semgs[b])

    def wait_gather(b):
        pltpu.make_async_copy(tok_hbm.at[idxc.at[b]], toks[b],
                              semgs[b]).wait()

    def wait_out(b):
        for s in range(LCH):
            pltpu.make_async_copy(
                obs[b].at[pl.ds(s * NBLK, NBLK)],
                out_hbm.at[0, pl.ds(0, NBLK)], semos[b]).wait()

    def compute(g, b):
        l0 = g * LCH
        tok = toks[b]
        ob = obs[b]
        c0 = c0s[b]
        for s in range(LCH):
            for k in range(KG):
                ksl = pl.ds(k * LANES, LANES)
                c0[s, ksl] = pos_v[l0 + s, ksl] + segt[0, ksl]
        for s in range(LCH):
            lvec = jnp.full((LANES,), l0 + s, dtype=jnp.int32)
            for h, nvec in enumerate(nvecs):
                svf = plsc.load_gather(sr, [nvec, lvec]).astype(jnp.float32)
                for j in range(LANES):
                    spl = jnp.full((LANES,), svf[j], dtype=jnp.float32)
                    r = s * NBLK + h * LANES + j
                    for k in range(KG):
                        ksl = pl.ds(k * LANES, LANES)
                        ob[r, ksl] = (tok[r, ksl] + c0[s, ksl]
                                      + spl * dline[0, ksl])

    def store(g, b):
        l0 = g * LCH
        for s in range(LCH):
            pltpu.async_copy(obs[b].at[pl.ds(s * NBLK, NBLK)],
                             out_hbm.at[l0 + s, pl.ds(n0, NBLK)], semos[b])

    issue(0, 0)
    issue(1, 1)

    def pair_body(go, carry):
        for b in (0, 1):
            g = 2 * go + b
            wait_gather(b)

            @pl.when(go > 0)
            def _():
                wait_out(b)

            compute(g, b)
            store(g, b)

            @pl.when(go < PAIRS - 1)
            def _():
                issue(g + 2, b)

        return carry

    lax.fori_loop(0, PAIRS, pair_body, 0)
    wait_out(0)
    wait_out(1)


def kernel(x, segments, token_table, segment_table, pos_embedding):
    pos = pos_embedding[:, 0, :]
    mesh = plsc.VectorSubcoreMesh(core_axis_name="c", subcore_axis_name="s")
    out = pl.kernel(
        _sc_body,
        out_type=jax.ShapeDtypeStruct((L, N, D), jnp.float32),
        mesh=mesh,
        scratch_types=[
            pltpu.VMEM((NBLK, L), jnp.int32),
            pltpu.VMEM((NBLK, L), jnp.int32),
            pltpu.VMEM((L, D), jnp.float32),
            pltpu.VMEM((2, D), jnp.float32),
            pltpu.VMEM((1, D), jnp.float32),
            pltpu.VMEM((2, CH_ROWS), jnp.int32),
            pltpu.VMEM((CH_ROWS, D), jnp.float32),
            pltpu.VMEM((CH_ROWS, D), jnp.float32),
            pltpu.VMEM((CH_ROWS, D), jnp.float32),
            pltpu.VMEM((CH_ROWS, D), jnp.float32),
            pltpu.VMEM((LCH, D), jnp.float32),
            pltpu.VMEM((LCH, D), jnp.float32),
            pltpu.SemaphoreType.DMA,
            pltpu.SemaphoreType.DMA,
            pltpu.SemaphoreType.DMA,
            pltpu.SemaphoreType.DMA,
        ],
        compiler_params=pltpu.CompilerParams(use_tc_tiling_on_sc=False,
                                             needs_layout_passes=False),
    )(x.astype(jnp.int32), segments.astype(jnp.int32), token_table,
      segment_table, pos)
    return out
